# baseline (speedup 1.0000x reference)
import jax
import jax.numpy as jnp
from jax import lax
from jax.experimental import pallas as pl
from jax.experimental.pallas import tpu as pltpu
from jax.experimental.pallas import tpu_sc as plsc

_C = 64
_M = 256
_N = 256
_T = 2048
_NC = 2
_NS = 16
_NW = _NC * _NS
_CHUNK = _T // _NW
_TT = 64
_NT = _T // _TT + _C // 2
_PAD = _NT * _TT


def _worker_id():
    return lax.axis_index("s") * _NC + lax.axis_index("c")


def _tcroute_body(inds_ref, slot_ref, texp_ref):
    iv = inds_ref[...].reshape(_T, 1)
    onehot = (lax.broadcasted_iota(jnp.int32, (_T, _C), 1) == iv
              ).astype(jnp.int32)
    incl = onehot
    k = 1
    while k < _T:
        shifted = jnp.concatenate(
            [jnp.zeros((k, _C), jnp.int32), incl[: _T - k, :]], axis=0)
        incl = incl + shifted
        k *= 2
    excl = incl - onehot
    totals = jnp.sum(onehot, axis=0, keepdims=True)
    tiles = (totals + (_TT - 1)) >> 6
    tincl = tiles
    k = 1
    while k < _C:
        shifted = jnp.concatenate(
            [jnp.zeros((1, k), jnp.int32), tincl[:, : _C - k]], axis=1)
        tincl = tincl + shifted
        k *= 2
    texcl = tincl - tiles
    base = texcl * _TT
    slot_ref[...] = jnp.sum(onehot * (excl + base), axis=1).reshape(_T)
    jv = lax.broadcasted_iota(jnp.int32, (_NT, _C), 0)
    texp_ref[...] = (jnp.sum(
        (jnp.broadcast_to(texcl, (_NT, _C)) <= jv).astype(jnp.int32),
        axis=1) - 1).reshape(_NT)


def _scatter_body(slot_hbm, x_hbm, xs_hbm, slot_v, xr_v, sem, sem2):
    wid = _worker_id()
    xcp = pltpu.async_copy(x_hbm.at[pl.ds(wid * _CHUNK, _CHUNK)], xr_v, sem)
    pltpu.async_copy(slot_hbm.at[pl.ds(wid * _CHUNK, _CHUNK)], slot_v,
                     sem2).wait()
    xcp.wait()
    pltpu.async_copy(xr_v, xs_hbm.at[slot_v], sem).wait()


_UNROLL = 16


def _mm_body(texp_ref, xs_ref, w_ref, b_ref, y_ref):
    i0 = pl.program_id(0) * _UNROLL
    for u in range(_UNROLL):
        e = texp_ref[i0 + u]
        xt = xs_ref[pl.ds(u * _TT, _TT), :].astype(jnp.bfloat16)
        wt = w_ref[e].astype(jnp.bfloat16)
        y_ref[pl.ds(u * _TT, _TT), :] = jnp.dot(
            xt, wt, preferred_element_type=jnp.float32) + b_ref[e]


def _unsort_body(slot_hbm, ys_hbm, out_hbm, slot_v, rows_v, sem):
    wid = _worker_id()
    pltpu.sync_copy(slot_hbm.at[pl.ds(wid * _CHUNK, _CHUNK)], slot_v)
    pltpu.async_copy(ys_hbm.at[slot_v], rows_v, sem).wait()
    pltpu.sync_copy(rows_v, out_hbm.at[pl.ds(wid * _CHUNK, _CHUNK)])


def kernel(x, inds, w, b):
    inds32 = inds.astype(jnp.int32)
    mesh = plsc.VectorSubcoreMesh(
        core_axis_name="c", subcore_axis_name="s",
        num_cores=_NC, num_subcores=_NS)

    slot, texp = pl.pallas_call(
        _tcroute_body,
        grid=(1,),
        in_specs=[pl.BlockSpec((_T,), lambda i: (0,))],
        out_specs=[
            pl.BlockSpec((_T,), lambda i: (0,)),
            pl.BlockSpec((_NT,), lambda i: (0,)),
        ],
        out_shape=(
            jax.ShapeDtypeStruct((_T,), jnp.int32),
            jax.ShapeDtypeStruct((_NT,), jnp.int32),
        ),
    )(inds32)

    scatter = pl.kernel(
        _scatter_body,
        out_type=jax.ShapeDtypeStruct((_PAD, _M), jnp.float32),
        mesh=mesh,
        compiler_params=pltpu.CompilerParams(needs_layout_passes=False),
        scratch_types=[
            pltpu.VMEM((_CHUNK,), jnp.int32),
            pltpu.VMEM((_CHUNK, _M), jnp.float32),
            pltpu.SemaphoreType.DMA,
            pltpu.SemaphoreType.DMA,
        ],
    )
    xs = scatter(slot, x)

    ys = pl.pallas_call(
        _mm_body,
        grid_spec=pltpu.PrefetchScalarGridSpec(
            num_scalar_prefetch=1,
            grid=(_NT // _UNROLL,),
            in_specs=[
                pl.BlockSpec((_UNROLL * _TT, _M), lambda i, te: (i, 0)),
                pl.BlockSpec((_C, _M, _N), lambda i, te: (0, 0, 0)),
                pl.BlockSpec((_C, 1, _N), lambda i, te: (0, 0, 0)),
            ],
            out_specs=pl.BlockSpec((_UNROLL * _TT, _N), lambda i, te: (i, 0)),
        ),
        out_shape=jax.ShapeDtypeStruct((_PAD, _N), jnp.float32),
    )(texp, xs, w, b)

    unsort = pl.kernel(
        _unsort_body,
        out_type=jax.ShapeDtypeStruct((_T, _N), jnp.float32),
        mesh=mesh,
        compiler_params=pltpu.CompilerParams(needs_layout_passes=False),
        scratch_types=[
            pltpu.VMEM((_CHUNK,), jnp.int32),
            pltpu.VMEM((_CHUNK, _N), jnp.float32),
            pltpu.SemaphoreType.DMA,
        ],
    )
    return unsort(slot, ys)
